# baseline (speedup 1.0000x reference)
import functools

import jax
import jax.numpy as jnp
from jax import lax
from jax.experimental import pallas as pl
from jax.experimental.pallas import tpu as pltpu
from jax.experimental.pallas import tpu_sc as plsc

NUM_ROWS = 1000000
B = 4096
L = 50
D = 64
O = 128
NC, NS = 2, 16
NW = NC * NS
BPW = B // NW

RC = 32768
YSPLIT = 425984
YREST = NUM_ROWS - YSPLIT
RPW = YSPLIT // NW
NBLK = RPW // BPW


def _repack_sc_body(t3_hbm, o2_hbm, slab_v, outv_v, isem, osem):
    wid = lax.axis_index("s") * NC + lax.axis_index("c")
    wbase = wid * RPW
    lanes = lax.iota(jnp.int32, 16)
    pltpu.async_copy(
        t3_hbm.at[:, :, pl.ds(wbase, BPW)],
        slab_v.at[:, :, pl.ds(0, BPW)],
        isem,
    )

    def body(k, carry):
        sl = lax.rem(k, 2)

        @pl.when(k + 1 < NBLK)
        def _prefetch():
            pltpu.async_copy(
                t3_hbm.at[:, :, pl.ds(wbase + (k + 1) * BPW, BPW)],
                slab_v.at[:, :, pl.ds((1 - sl) * BPW, BPW)],
                isem,
            )

        pltpu.make_async_copy(
            t3_hbm.at[:, :, pl.ds(0, BPW)],
            slab_v.at[:, :, pl.ds(0, BPW)],
            isem,
        ).wait()

        @pl.when(k >= 2)
        def _free_out():
            pltpu.make_async_copy(
                outv_v.at[0], o2_hbm.at[pl.ds(0, BPW)], osem
            ).wait()

        slvec = jnp.full((16,), sl, jnp.int32)
        for m in range(8):
            for s in range(8):
                c = jnp.full((16,), 8 * m + s, jnp.int32)
                for g in range(BPW // 16):
                    v = slab_v[m, s, pl.ds(sl * BPW + g * 16, 16)]
                    plsc.store_scatter(
                        outv_v, [slvec, lanes + g * 16, c], v
                    )
        pltpu.async_copy(
            outv_v.at[sl], o2_hbm.at[pl.ds(wbase + k * BPW, BPW)], osem
        )
        return carry

    lax.fori_loop(0, NBLK, body, 0)
    for _ in range(2):
        pltpu.make_async_copy(
            outv_v.at[0], o2_hbm.at[pl.ds(0, BPW)], osem
        ).wait()


_repack_sc = functools.partial(
    pl.kernel,
    out_type=jax.ShapeDtypeStruct((YSPLIT, 2 * D), jnp.float32),
    mesh=plsc.VectorSubcoreMesh(core_axis_name="c", subcore_axis_name="s"),
    scratch_types=[
        pltpu.VMEM((8, 8, 2 * BPW), jnp.float32),
        pltpu.VMEM((2, BPW, 2 * D), jnp.float32),
        pltpu.SemaphoreType.DMA,
        pltpu.SemaphoreType.DMA,
    ],
    compiler_params=pltpu.CompilerParams(needs_layout_passes=False),
)(_repack_sc_body)


def _repack_body(t_ref, o_ref):
    o_ref[:, 0:D] = jnp.transpose(t_ref[...])


def _repack(t):
    grid = (YREST + RC - 1) // RC
    return pl.pallas_call(
        _repack_body,
        out_shape=jax.ShapeDtypeStruct((YREST, 2 * D), jnp.float32),
        grid=(grid,),
        in_specs=[pl.BlockSpec((D, RC), lambda i: (0, YSPLIT // RC + i))],
        out_specs=pl.BlockSpec((RC, 2 * D), lambda i: (i, 0)),
    )(t)


def _pool_body(vt_hbm, t1_hbm, t2_hbm, out_hbm, cnt_hbm,
               raw_v, idx1_v, idx2_v, acc_v, cnt_v, sem, isem):
    wid = lax.axis_index("s") * NC + lax.axis_index("c")
    base = wid * BPW
    idx_cp = pltpu.async_copy(vt_hbm.at[pl.ds(base, BPW)], raw_v, isem)

    zeros = jnp.zeros((16,), jnp.float32)

    def zbody(i, carry):
        for g in range(D // 16):
            acc_v[i, pl.ds(g * 16, 16)] = zeros
        return carry

    lax.fori_loop(0, BPW, zbody, 0)
    idx_cp.wait()

    lanes = lax.iota(jnp.int32, 16)
    yvec = jnp.full((16,), YSPLIT, jnp.int32)
    izeros = jnp.zeros((16,), jnp.int32)

    for g in range(BPW // 16):
        rows = lanes + g * 16

        def tbody(j, cnt16, rows=rows):
            cols = jnp.full((16,), j, jnp.int32)
            v = plsc.load_gather(raw_v, [rows, cols])
            m = v < yvec
            w = v - yvec
            idx1_v[j, pl.ds(g * 16, 16)] = jnp.where(m, izeros, w + w)
            idx2_v[j, pl.ds(g * 16, 16)] = jnp.where(m, v + v, izeros)
            return cnt16 + jnp.where(m, 1.0, 0.0).astype(jnp.float32)

        cnt16 = lax.fori_loop(0, L, tbody, zeros)
        cnt_v[pl.ds(g * 16, 16)] = cnt16

    def fire(j, carry):
        pltpu.async_copy(t1_hbm.at[idx1_v.at[j]], acc_v, sem, add=True)
        pltpu.async_copy(t2_hbm.at[idx2_v.at[j]], acc_v, sem, add=True)
        return carry

    lax.fori_loop(0, L, fire, 0)

    def drain(j, carry):
        pltpu.make_async_copy(t1_hbm.at[idx1_v.at[0]], acc_v, sem).wait()
        pltpu.make_async_copy(t2_hbm.at[idx2_v.at[0]], acc_v, sem).wait()
        return carry

    lax.fori_loop(0, L, drain, 0)
    pltpu.sync_copy(acc_v, out_hbm.at[pl.ds(base, BPW)])
    pltpu.sync_copy(cnt_v, cnt_hbm.at[pl.ds(base, BPW)])


_pool = functools.partial(
    pl.kernel,
    out_type=[
        jax.ShapeDtypeStruct((B, D), jnp.float32),
        jax.ShapeDtypeStruct((B,), jnp.float32),
    ],
    mesh=plsc.VectorSubcoreMesh(core_axis_name="c", subcore_axis_name="s"),
    scratch_types=[
        pltpu.VMEM((BPW, L), jnp.int32),
        pltpu.VMEM((L, BPW), jnp.int32),
        pltpu.VMEM((L, BPW), jnp.int32),
        pltpu.VMEM((BPW, D), jnp.float32),
        pltpu.VMEM((BPW,), jnp.float32),
        pltpu.SemaphoreType.DMA,
        pltpu.SemaphoreType.DMA,
    ],
    compiler_params=pltpu.CompilerParams(
        use_tc_tiling_on_sc=False, needs_layout_passes=False
    ),
)(_pool_body)


def _linear_body(x_ref, cnt_ref, ty_ref, t0_ref, w_ref, b_ref, o_ref):
    cnt = cnt_ref[...]
    x = (
        x_ref[...]
        - cnt * ty_ref[...]
        - (jnp.float32(L) - cnt) * t0_ref[...]
    ) * jnp.float32(1.0 / L)
    o_ref[...] = (
        lax.dot_general(
            x, w_ref[...], (((1,), (1,)), ((), ())),
            preferred_element_type=jnp.float32,
        )
        + b_ref[...]
    )


def _linear(pooled, cnt, ty_row, t0_row, W, b2d):
    blk = 512
    return pl.pallas_call(
        _linear_body,
        out_shape=jax.ShapeDtypeStruct((B, O), jnp.float32),
        grid=(B // blk,),
        in_specs=[
            pl.BlockSpec((blk, D), lambda i: (i, 0)),
            pl.BlockSpec((blk, 1), lambda i: (i, 0)),
            pl.BlockSpec((1, D), lambda i: (0, 0)),
            pl.BlockSpec((1, D), lambda i: (0, 0)),
            pl.BlockSpec((O, D), lambda i: (0, 0)),
            pl.BlockSpec((1, O), lambda i: (0, 0)),
        ],
        out_specs=pl.BlockSpec((blk, O), lambda i: (i, 0)),
    )(pooled, cnt, ty_row, t0_row, W, b2d)


def kernel(vectorized_text, emb_table, W, b):
    t = emb_table.T
    o1 = _repack(t)
    o2 = _repack_sc(t.reshape(8, 8, NUM_ROWS))
    tb1 = o1.reshape(2 * YREST, D)
    tb2 = o2.reshape(2 * YSPLIT, D)
    pooled, cnt = _pool(vectorized_text.astype(jnp.int32), tb1, tb2)
    return _linear(
        pooled, cnt.reshape(B, 1), tb1[0:1], tb2[0:1], W, b.reshape(1, O)
    )


# speedup vs baseline: 2.6880x; 2.6880x over previous
import functools

import jax
import jax.numpy as jnp
from jax import lax
from jax.experimental import pallas as pl
from jax.experimental.pallas import tpu as pltpu
from jax.experimental.pallas import tpu_sc as plsc

NUM_ROWS = 1000000
B = 4096
L = 50
D = 64
O = 128
NC, NS = 2, 16
NW = NC * NS
BPW = B // NW


def _pool_body(vt_hbm, table_hbm, out_hbm, raw_v, idx_v, acc_v, sem, isem):
    wid = lax.axis_index("s") * NC + lax.axis_index("c")
    base = wid * BPW
    idx_cp = pltpu.async_copy(vt_hbm.at[pl.ds(base, BPW)], raw_v, isem)

    zeros = jnp.zeros((32,), jnp.bfloat16)

    def zbody(i, carry):
        for g in range(D // 32):
            acc_v[i, pl.ds(g * 32, 32)] = zeros
        return carry

    lax.fori_loop(0, BPW, zbody, 0)
    idx_cp.wait()

    lanes = lax.iota(jnp.int32, 16)

    def tbody(j, carry):
        cols = jnp.full((16,), j, jnp.int32)
        for g in range(BPW // 16):
            v = plsc.load_gather(raw_v, [lanes + g * 16, cols])
            idx_v[j, pl.ds(g * 16, 16)] = v + v
        return carry

    lax.fori_loop(0, L, tbody, 0)

    def fire(j, carry):
        pltpu.async_copy(table_hbm.at[idx_v.at[j]], acc_v, sem, add=True)
        return carry

    lax.fori_loop(0, L, fire, 0)

    def drain(j, carry):
        pltpu.make_async_copy(table_hbm.at[idx_v.at[0]], acc_v, sem).wait()
        return carry

    lax.fori_loop(0, L, drain, 0)
    pltpu.sync_copy(acc_v, out_hbm.at[pl.ds(base, BPW)])


_pool = functools.partial(
    pl.kernel,
    out_type=jax.ShapeDtypeStruct((B, D), jnp.bfloat16),
    mesh=plsc.VectorSubcoreMesh(core_axis_name="c", subcore_axis_name="s"),
    scratch_types=[
        pltpu.VMEM((BPW, L), jnp.int32),
        pltpu.VMEM((L, BPW), jnp.int32),
        pltpu.VMEM((BPW, D), jnp.bfloat16),
        pltpu.SemaphoreType.DMA,
        pltpu.SemaphoreType.DMA,
    ],
    compiler_params=pltpu.CompilerParams(
        use_tc_tiling_on_sc=False, needs_layout_passes=False
    ),
)(_pool_body)


RC = 32768


def _repack_body(t_ref, o_ref):
    o_ref[:, 0:D] = jnp.transpose(t_ref[...].astype(jnp.bfloat16))


def _repack(t):
    grid = (NUM_ROWS + RC - 1) // RC
    return pl.pallas_call(
        _repack_body,
        out_shape=jax.ShapeDtypeStruct((NUM_ROWS, 2 * D), jnp.bfloat16),
        grid=(grid,),
        in_specs=[pl.BlockSpec((D, RC), lambda i: (0, i))],
        out_specs=pl.BlockSpec((RC, 2 * D), lambda i: (i, 0)),
    )(t)


def _linear_body(x_ref, w_ref, b_ref, o_ref):
    x = x_ref[...].astype(jnp.float32) * jnp.float32(1.0 / L)
    o_ref[...] = (
        lax.dot_general(
            x, w_ref[...], (((1,), (1,)), ((), ())),
            preferred_element_type=jnp.float32,
        )
        + b_ref[...]
    )


def _linear(pooled, W, b2d):
    blk = 512
    return pl.pallas_call(
        _linear_body,
        out_shape=jax.ShapeDtypeStruct((B, O), jnp.float32),
        grid=(B // blk,),
        in_specs=[
            pl.BlockSpec((blk, D), lambda i: (i, 0)),
            pl.BlockSpec((O, D), lambda i: (0, 0)),
            pl.BlockSpec((1, O), lambda i: (0, 0)),
        ],
        out_specs=pl.BlockSpec((blk, O), lambda i: (i, 0)),
    )(pooled, W, b2d)


def kernel(vectorized_text, emb_table, W, b):
    packed = _repack(emb_table.T)
    pooled = _pool(
        vectorized_text.astype(jnp.int32),
        packed.reshape(2 * NUM_ROWS, D),
    )
    return _linear(pooled, W, b.reshape(1, O))


# speedup vs baseline: 9.6903x; 3.6051x over previous
import functools

import jax
import jax.numpy as jnp
from jax import lax
from jax.experimental import pallas as pl
from jax.experimental.pallas import tpu as pltpu
from jax.experimental.pallas import tpu_sc as plsc

NUM_ROWS = 1000000
B = 4096
L = 50
D = 64
O = 128
NC, NS = 2, 16
NW = NC * NS
BPW = B // NW


def _pool_body(vt_hbm, table_hbm, out_hbm, raw_v, idx_v, acc_v, sem, isem):
    wid = lax.axis_index("s") * NC + lax.axis_index("c")
    base = wid * BPW
    idx_cp = pltpu.async_copy(vt_hbm.at[pl.ds(base, BPW)], raw_v, isem)

    zeros = jnp.zeros((16,), jnp.float32)

    def zbody(i, carry):
        for g in range(D // 16):
            acc_v[i, pl.ds(g * 16, 16)] = zeros
        return carry

    lax.fori_loop(0, BPW, zbody, 0)
    idx_cp.wait()

    lanes = lax.iota(jnp.int32, 16)

    def tbody(j, carry):
        cols = jnp.full((16,), j, jnp.int32)
        for g in range(BPW // 16):
            v = plsc.load_gather(raw_v, [lanes + g * 16, cols])
            idx_v[j, pl.ds(g * 16, 16)] = v + v
        return carry

    lax.fori_loop(0, L, tbody, 0)

    def fire(j, carry):
        pltpu.async_copy(table_hbm.at[idx_v.at[j]], acc_v, sem, add=True)
        return carry

    lax.fori_loop(0, L, fire, 0)

    def drain(j, carry):
        pltpu.make_async_copy(table_hbm.at[idx_v.at[0]], acc_v, sem).wait()
        return carry

    lax.fori_loop(0, L, drain, 0)
    pltpu.sync_copy(acc_v, out_hbm.at[pl.ds(base, BPW)])


_pool = functools.partial(
    pl.kernel,
    out_type=jax.ShapeDtypeStruct((B, D), jnp.float32),
    mesh=plsc.VectorSubcoreMesh(core_axis_name="c", subcore_axis_name="s"),
    scratch_types=[
        pltpu.VMEM((BPW, L), jnp.int32),
        pltpu.VMEM((L, BPW), jnp.int32),
        pltpu.VMEM((BPW, D), jnp.float32),
        pltpu.SemaphoreType.DMA,
        pltpu.SemaphoreType.DMA,
    ],
    compiler_params=pltpu.CompilerParams(
        use_tc_tiling_on_sc=False, needs_layout_passes=False
    ),
)(_pool_body)


RC = 32768


def _repack_body(t_ref, o_ref):
    o_ref[:, 0:D] = jnp.transpose(t_ref[...])


def _repack(t):
    grid = (NUM_ROWS + RC - 1) // RC
    return pl.pallas_call(
        _repack_body,
        out_shape=jax.ShapeDtypeStruct((NUM_ROWS, 2 * D), jnp.float32),
        grid=(grid,),
        in_specs=[pl.BlockSpec((D, RC), lambda i: (0, i))],
        out_specs=pl.BlockSpec((RC, 2 * D), lambda i: (i, 0)),
    )(t)


def _linear_body(x_ref, w_ref, b_ref, o_ref):
    x = x_ref[...] * jnp.float32(1.0 / L)
    o_ref[...] = (
        lax.dot_general(
            x, w_ref[...], (((1,), (1,)), ((), ())),
            preferred_element_type=jnp.float32,
        )
        + b_ref[...]
    )


def _linear(pooled, W, b2d):
    blk = 512
    return pl.pallas_call(
        _linear_body,
        out_shape=jax.ShapeDtypeStruct((B, O), jnp.float32),
        grid=(B // blk,),
        in_specs=[
            pl.BlockSpec((blk, D), lambda i: (i, 0)),
            pl.BlockSpec((O, D), lambda i: (0, 0)),
            pl.BlockSpec((1, O), lambda i: (0, 0)),
        ],
        out_specs=pl.BlockSpec((blk, O), lambda i: (i, 0)),
    )(pooled, W, b2d)


def kernel(vectorized_text, emb_table, W, b):
    packed = _repack(emb_table.T)
    pooled = _pool(
        vectorized_text.astype(jnp.int32),
        packed.reshape(2 * NUM_ROWS, D),
    )
    return _linear(pooled, W, b.reshape(1, O))


# speedup vs baseline: 9.7487x; 1.0060x over previous
import functools

import jax
import jax.numpy as jnp
from jax import lax
from jax.experimental import pallas as pl
from jax.experimental.pallas import tpu as pltpu
from jax.experimental.pallas import tpu_sc as plsc

NUM_ROWS = 1000000
B = 4096
L = 50
D = 64
O = 128
NC, NS = 2, 16
NW = NC * NS
BPW = B // NW


def _pool_body(vt_hbm, table_hbm, out_hbm, raw_v, idx_v, acc_v, sem, isem):
    wid = lax.axis_index("s") * NC + lax.axis_index("c")
    base = wid * BPW
    idx_cp = pltpu.async_copy(vt_hbm.at[pl.ds(base, BPW)], raw_v, isem)

    zeros = jnp.zeros((16,), jnp.float32)

    def zbody(i, carry):
        for g in range(D // 16):
            acc_v[i, pl.ds(g * 16, 16)] = zeros
        return carry

    lax.fori_loop(0, BPW, zbody, 0)
    idx_cp.wait()

    lanes = lax.iota(jnp.int32, 16)

    def tbody(j, carry):
        cols = jnp.full((16,), j, jnp.int32)
        for g in range(BPW // 16):
            v = plsc.load_gather(raw_v, [lanes + g * 16, cols])
            idx_v[j, pl.ds(g * 16, 16)] = v + v
        return carry

    lax.fori_loop(0, L, tbody, 0)

    def fire(j, carry):
        pltpu.async_copy(table_hbm.at[idx_v.at[j]], acc_v, sem, add=True)
        return carry

    lax.fori_loop(0, L, fire, 0)

    def drain(j, carry):
        pltpu.make_async_copy(table_hbm.at[idx_v.at[0]], acc_v, sem).wait()
        return carry

    lax.fori_loop(0, L, drain, 0)
    pltpu.sync_copy(acc_v, out_hbm.at[pl.ds(base, BPW)])


_pool = functools.partial(
    pl.kernel,
    out_type=jax.ShapeDtypeStruct((B, D), jnp.float32),
    mesh=plsc.VectorSubcoreMesh(core_axis_name="c", subcore_axis_name="s"),
    scratch_types=[
        pltpu.VMEM((BPW, L), jnp.int32),
        pltpu.VMEM((L, BPW), jnp.int32),
        pltpu.VMEM((BPW, D), jnp.float32),
        pltpu.SemaphoreType.DMA,
        pltpu.SemaphoreType.DMA,
    ],
    compiler_params=pltpu.CompilerParams(
        use_tc_tiling_on_sc=False, needs_layout_passes=False
    ),
)(_pool_body)


RC = 40960


def _repack_body(t_ref, o_ref):
    o_ref[:, 0:D] = jnp.transpose(t_ref[...])


def _repack(t):
    grid = (NUM_ROWS + RC - 1) // RC
    return pl.pallas_call(
        _repack_body,
        out_shape=jax.ShapeDtypeStruct((NUM_ROWS, 2 * D), jnp.float32),
        grid=(grid,),
        in_specs=[pl.BlockSpec((D, RC), lambda i: (0, i))],
        out_specs=pl.BlockSpec((RC, 2 * D), lambda i: (i, 0)),
        compiler_params=pltpu.CompilerParams(
            vmem_limit_bytes=120 * 1024 * 1024
        ),
    )(t)


def _linear_body(x_ref, w_ref, b_ref, o_ref):
    x = x_ref[...] * jnp.float32(1.0 / L)
    o_ref[...] = (
        lax.dot_general(
            x, w_ref[...], (((1,), (1,)), ((), ())),
            preferred_element_type=jnp.float32,
        )
        + b_ref[...]
    )


def _linear(pooled, W, b2d):
    blk = 512
    return pl.pallas_call(
        _linear_body,
        out_shape=jax.ShapeDtypeStruct((B, O), jnp.float32),
        grid=(B // blk,),
        in_specs=[
            pl.BlockSpec((blk, D), lambda i: (i, 0)),
            pl.BlockSpec((O, D), lambda i: (0, 0)),
            pl.BlockSpec((1, O), lambda i: (0, 0)),
        ],
        out_specs=pl.BlockSpec((blk, O), lambda i: (i, 0)),
    )(pooled, W, b2d)


def kernel(vectorized_text, emb_table, W, b):
    packed = _repack(emb_table.T)
    pooled = _pool(
        vectorized_text.astype(jnp.int32),
        packed.reshape(2 * NUM_ROWS, D),
    )
    return _linear(pooled, W, b.reshape(1, O))
